# Initial kernel scaffold; baseline (speedup 1.0000x reference)
import jax
import jax.numpy as jnp
from jax.experimental import pallas as pl


def kernel(img, label, We, be, Wd, bd):
    raise NotImplementedError("write your pallas kernel here")


# speedup vs baseline: 2.1557x; 2.1557x over previous
import functools

import jax
import jax.numpy as jnp
from jax import lax
from jax.experimental import pallas as pl
from jax.experimental.pallas import tpu as pltpu
from jax.experimental.pallas import tpu_sc as plsc

E = 8
D = 768
H = 128
N = 16384
BLK = 256
NB = N // BLK + E
NPAD = NB * BLK

RG = 128
RC = N // RG


def _route_body(lab_ref, dest_ref, blkexp_ref, limit_ref):
    lab = lab_ref[...]
    io0 = lax.broadcasted_iota(jnp.int32, (RC, RC), 0)
    io1 = lax.broadcasted_iota(jnp.int32, (RC, RC), 1)
    m_cols = (io0 < io1).astype(jnp.float32)
    l_rows = (io1 < io0).astype(jnp.float32)

    counts = []
    offs = []
    dest = jnp.zeros((RG, RC), dtype=jnp.int32)
    off = jnp.int32(0)
    for e in range(E):
        mask = (lab == e)
        maskf = mask.astype(jnp.float32)
        within = jnp.dot(maskf, m_cols, preferred_element_type=jnp.float32)
        rowcnt = jnp.sum(maskf, axis=1, keepdims=True)
        rowpre = jnp.dot(l_rows, rowcnt,
                         preferred_element_type=jnp.float32)
        rank = (within + rowpre).astype(jnp.int32)
        cnt = jnp.sum(mask.astype(jnp.int32))
        counts.append(cnt)
        offs.append(off)
        dest = dest + jnp.where(mask, off + rank, 0)
        padded = ((cnt + BLK - 1) // BLK) * BLK
        off = off + padded
    dest_ref[...] = dest

    brow = lax.broadcasted_iota(jnp.int32, (1, NB), 1) * BLK
    blkexp = jnp.zeros((1, NB), dtype=jnp.int32)
    limit = jnp.zeros((1, NB), dtype=jnp.int32)
    for e in range(E):
        lo = offs[e]
        hi = offs[e + 1] if e + 1 < E else off
        ind = (brow >= lo) & (brow < hi)
        blkexp = blkexp + jnp.where(ind, e, 0)
        limit = limit + jnp.where(ind, lo + counts[e], 0)
    blkexp_ref[...] = blkexp
    limit_ref[...] = limit


def _route(label2d):
    return pl.pallas_call(
        _route_body,
        out_shape=(
            jax.ShapeDtypeStruct((RG, RC), jnp.int32),
            jax.ShapeDtypeStruct((1, NB), jnp.int32),
            jax.ShapeDtypeStruct((1, NB), jnp.int32),
        ),
    )(label2d)


_SC_CH = 128


def _sc_scatter(img, dest3d):
    mesh = plsc.VectorSubcoreMesh(core_axis_name="c", subcore_axis_name="s")
    nw = mesh.num_cores * mesh.num_subcores
    tpw = N // nw
    nch = tpw // _SC_CH

    @functools.partial(
        pl.kernel,
        out_type=jax.ShapeDtypeStruct((NPAD, D), jnp.float32),
        mesh=mesh,
        scratch_types=[
            pltpu.VMEM((nch, _SC_CH), jnp.int32),
            pltpu.VMEM((_SC_CH, D), jnp.float32),
            pltpu.SemaphoreType.DMA,
        ],
    )
    def k(img_hbm, dest_hbm, xpad_hbm, idx_v, rows_v, sem):
        wid = lax.axis_index("s") * mesh.num_cores + lax.axis_index("c")
        base = wid * tpw
        pltpu.sync_copy(dest_hbm.at[wid], idx_v)
        for j in range(nch):
            pltpu.sync_copy(img_hbm.at[pl.ds(base + j * _SC_CH, _SC_CH)],
                            rows_v)
            pltpu.async_copy(rows_v, xpad_hbm.at[idx_v.at[j]], sem).wait()

    return k(img, dest3d)


def _sc_gather(ypad, dest3d):
    mesh = plsc.VectorSubcoreMesh(core_axis_name="c", subcore_axis_name="s")
    nw = mesh.num_cores * mesh.num_subcores
    tpw = N // nw
    nch = tpw // _SC_CH

    @functools.partial(
        pl.kernel,
        out_type=jax.ShapeDtypeStruct((N, D), jnp.float32),
        mesh=mesh,
        scratch_types=[
            pltpu.VMEM((nch, _SC_CH), jnp.int32),
            pltpu.VMEM((_SC_CH, D), jnp.float32),
            pltpu.SemaphoreType.DMA,
        ],
    )
    def k(ypad_hbm, dest_hbm, out_hbm, idx_v, rows_v, sem):
        wid = lax.axis_index("s") * mesh.num_cores + lax.axis_index("c")
        base = wid * tpw
        pltpu.sync_copy(dest_hbm.at[wid], idx_v)
        for j in range(nch):
            pltpu.async_copy(ypad_hbm.at[idx_v.at[j]], rows_v, sem).wait()
            pltpu.sync_copy(rows_v, out_hbm.at[pl.ds(base + j * _SC_CH, _SC_CH)])

    return k(ypad, dest3d)


def _expert_body(be_idx_ref, lim_ref, x_ref, we_ref, bee_ref, wd_ref, bd_ref,
                 y_ref, loss_ref, acc_ref):
    b = pl.program_id(0)
    x = x_ref[...]
    h = jnp.dot(x, we_ref[0], preferred_element_type=jnp.float32)
    h = h + bee_ref[0]
    y = jnp.dot(h, wd_ref[0], preferred_element_type=jnp.float32)
    y = y + bd_ref[0]
    y_ref[...] = y

    limit = lim_ref[0, b]
    row = b * BLK + lax.broadcasted_iota(jnp.int32, (BLK, 1), 0)
    diff = y - x
    sq = jnp.where(row < limit, diff * diff, 0.0)

    @pl.when(b == 0)
    def _():
        acc_ref[0] = 0.0

    acc_ref[0] += jnp.sum(sq)

    @pl.when(b == NB - 1)
    def _():
        loss_ref[...] = jnp.reshape(acc_ref[0] / (N * D), (1, 1))


def _experts(xpad, We, be, Wd, bd, blkexp, limit):
    grid_spec = pltpu.PrefetchScalarGridSpec(
        num_scalar_prefetch=2,
        grid=(NB,),
        in_specs=[
            pl.BlockSpec((BLK, D), lambda b, bexp, lim: (b, 0)),
            pl.BlockSpec((1, D, H), lambda b, bexp, lim: (bexp[0, b], 0, 0)),
            pl.BlockSpec((1, 1, H), lambda b, bexp, lim: (bexp[0, b], 0, 0)),
            pl.BlockSpec((1, H, D), lambda b, bexp, lim: (bexp[0, b], 0, 0)),
            pl.BlockSpec((1, 1, D), lambda b, bexp, lim: (bexp[0, b], 0, 0)),
        ],
        out_specs=[
            pl.BlockSpec((BLK, D), lambda b, bexp, lim: (b, 0)),
            pl.BlockSpec((1, 1), lambda b, bexp, lim: (0, 0)),
        ],
        scratch_shapes=[pltpu.SMEM((1,), jnp.float32)],
    )
    return pl.pallas_call(
        _expert_body,
        grid_spec=grid_spec,
        out_shape=(
            jax.ShapeDtypeStruct((NPAD, D), jnp.float32),
            jax.ShapeDtypeStruct((1, 1), jnp.float32),
        ),
    )(blkexp, limit, xpad, We, be.reshape(E, 1, H), Wd, bd.reshape(E, 1, D))


def kernel(img, label, We, be, Wd, bd):
    label2d = label.astype(jnp.int32).reshape(RG, RC)
    dest, blkexp, limit = _route(label2d)
    mesh = plsc.VectorSubcoreMesh(core_axis_name="c", subcore_axis_name="s")
    nw = mesh.num_cores * mesh.num_subcores
    dest3d = dest.reshape(nw, (N // nw) // _SC_CH, _SC_CH)
    xpad = _sc_scatter(img, dest3d)
    ypad, loss = _experts(xpad, We, be, Wd, bd, blkexp, limit)
    out = _sc_gather(ypad, dest3d)
    return loss.reshape(()), out
